# baseline (speedup 1.0000x reference)
import functools

import jax
import jax.numpy as jnp
from jax import lax
from jax.experimental import pallas as pl
from jax.experimental.pallas import tpu as pltpu
from jax.experimental.pallas import tpu_sc as plsc

_NE = 1024
_D = 64
_B = 32
_BR = 1024
_NROWS = _B * _BR
_NC, _NS = 2, 16
_L = 16
_DCH = 16


_BB = 2


def _tc_body(xt_ref, e_ref, idx_ref, loss_ref):
    i = pl.program_id(0)
    part = 0.0
    for j in range(_BB):
        xt = xt_ref[j]
        ssq = jnp.sum(xt * xt, axis=0, keepdims=True)
        norm = jnp.sqrt(ssq)
        inv = 1.0 / jnp.maximum(norm, 1e-12)
        xnt = xt * inv

        s = lax.dot_general(e_ref[...], xnt, (((1,), (0,)), ((), ())),
                            preferred_element_type=jnp.float32)
        m = jnp.max(s, axis=0, keepdims=True)
        idx = jnp.argmax(s, axis=0)[None].astype(jnp.int32)
        idx_ref[j] = idx

        xnsq = ssq * (inv * inv)
        part += jnp.sum(xnsq - 2.0 * m + 1.0) * (1.0 / (_NROWS * _D))

    @pl.when(i == 0)
    def _():
        loss_ref[0, 0] = 0.0

    loss_ref[0, 0] += part


_tc_call = pl.pallas_call(
    _tc_body,
    grid=(_B // _BB,),
    in_specs=[
        pl.BlockSpec((_BB, _D, _BR), lambda i: (i, 0, 0)),
        pl.BlockSpec((_NE, _D), lambda i: (0, 0)),
    ],
    out_specs=[
        pl.BlockSpec((_BB, 1, _BR), lambda i: (i, 0, 0)),
        pl.BlockSpec(memory_space=pltpu.SMEM, block_shape=(1, 1),
                     index_map=lambda i: (0, 0)),
    ],
    out_shape=[
        jax.ShapeDtypeStruct((_B, 1, _BR), jnp.int32),
        jax.ShapeDtypeStruct((1, 1), jnp.float32),
    ],
    compiler_params=pltpu.CompilerParams(dimension_semantics=("arbitrary",)),
)


@functools.cache
def _sc_gather_call():
    @functools.partial(
        pl.kernel,
        mesh=plsc.VectorSubcoreMesh(core_axis_name="c", subcore_axis_name="s"),
        out_type=jax.ShapeDtypeStruct((_B, _D * _BR), jnp.float32),
        scratch_types=[
            pltpu.VMEM((_D * _NE,), jnp.float32),
            pltpu.VMEM((_BR,), jnp.int32),
            pltpu.VMEM((_DCH * _BR,), jnp.float32),
        ],
        compiler_params=pltpu.CompilerParams(use_tc_tiling_on_sc=False,
                                             needs_layout_passes=False),
    )
    def _sc_gather(etf_hbm, idx_hbm, out_hbm, et_v, idx_v, q_v):
        b = lax.axis_index("s") * _NC + lax.axis_index("c")
        pltpu.sync_copy(etf_hbm, et_v)
        pltpu.sync_copy(idx_hbm.at[b], idx_v)

        for c in range(_D // _DCH):
            @plsc.parallel_loop(0, _BR // _L, 1, unroll=16)
            def _(r, c=c):
                base = r * _L
                ct = base // 128
                cc = base % 128
                cols = idx_v[pl.ds(base, _L)]
                for dd in range(_DCH):
                    d = c * _DCH + dd
                    off = (dd // 8) * 8192 + (dd % 8) * 128 + cc
                    q_v[pl.ds(ct * 1024 + off, _L)] = plsc.load_gather(
                        et_v, [cols + d * _NE])

            pltpu.sync_copy(q_v, out_hbm.at[b, pl.ds(c * _DCH * _BR,
                                                     _DCH * _BR)])

    return _sc_gather


def kernel(inputs, embeddings):
    orig_shape = inputs.shape
    xt3 = jnp.swapaxes(inputs, 1, 2)
    idx3, loss11 = _tc_call(xt3, embeddings)
    idx2d = idx3.reshape(_B, _BR)
    etf = embeddings.T.reshape(-1)
    qt = _sc_gather_call()(etf, idx2d)
    quantized = jnp.transpose(qt.reshape(_B, 8, 8, 8, 128),
                              (0, 2, 4, 1, 3)).reshape(orig_shape)
    loss = loss11[0, 0]
    encoding_indices = idx2d.reshape(orig_shape[:-1])
    return (quantized, loss, encoding_indices)


# speedup vs baseline: 1.1304x; 1.1304x over previous
import functools

import jax
import jax.numpy as jnp
from jax import lax
from jax.experimental import pallas as pl
from jax.experimental.pallas import tpu as pltpu
from jax.experimental.pallas import tpu_sc as plsc

_NE = 1024
_D = 64
_B = 32
_BR = 1024
_NROWS = _B * _BR
_NC, _NS = 2, 16
_L = 16
_DCH = 16


_BB = 2


def _tc_body(xt_ref, e_ref, idx_ref, loss_ref):
    i = pl.program_id(0)
    part = 0.0
    for j in range(_BB):
        xt = xt_ref[j]
        ssq = jnp.sum(xt * xt, axis=0, keepdims=True)
        norm = jnp.sqrt(ssq)
        inv = 1.0 / jnp.maximum(norm, 1e-12)
        xnt = xt * inv

        s = lax.dot_general(e_ref[...], xnt, (((1,), (0,)), ((), ())),
                            preferred_element_type=jnp.float32)
        m = jnp.max(s, axis=0, keepdims=True)
        idx = jnp.argmax(s, axis=0)[None].astype(jnp.int32)
        idx_ref[j] = idx

        xnsq = ssq * (inv * inv)
        part += jnp.sum(xnsq - 2.0 * m + 1.0) * (1.0 / (_NROWS * _D))

    @pl.when(i == 0)
    def _():
        loss_ref[0, 0] = 0.0

    loss_ref[0, 0] += part


_tc_call = pl.pallas_call(
    _tc_body,
    grid=(_B // _BB,),
    in_specs=[
        pl.BlockSpec((_BB, _D, _BR), lambda i: (i, 0, 0)),
        pl.BlockSpec((_NE, _D), lambda i: (0, 0)),
    ],
    out_specs=[
        pl.BlockSpec((_BB, 1, _BR), lambda i: (i, 0, 0)),
        pl.BlockSpec(memory_space=pltpu.SMEM, block_shape=(1, 1),
                     index_map=lambda i: (0, 0)),
    ],
    out_shape=[
        jax.ShapeDtypeStruct((_B, 1, _BR), jnp.int32),
        jax.ShapeDtypeStruct((1, 1), jnp.float32),
    ],
    compiler_params=pltpu.CompilerParams(dimension_semantics=("arbitrary",)),
)


@functools.cache
def _sc_gather_call():
    @functools.partial(
        pl.kernel,
        mesh=plsc.VectorSubcoreMesh(core_axis_name="c", subcore_axis_name="s"),
        out_type=jax.ShapeDtypeStruct((_B, _D * _BR), jnp.float32),
        scratch_types=[
            pltpu.VMEM((_D * _NE,), jnp.float32),
            pltpu.VMEM((_BR,), jnp.int32),
            pltpu.VMEM((_DCH * _BR,), jnp.float32),
        ],
        compiler_params=pltpu.CompilerParams(use_tc_tiling_on_sc=False,
                                             needs_layout_passes=False),
    )
    def _sc_gather(etf_hbm, idx_hbm, out_hbm, et_v, idx_v, q_v):
        b = lax.axis_index("s") * _NC + lax.axis_index("c")
        pltpu.sync_copy(etf_hbm, et_v)
        pltpu.sync_copy(idx_hbm.at[b], idx_v)

        for c in range(_D // _DCH):
            @plsc.parallel_loop(0, _BR // _L, 1, unroll=8)
            def _(r, c=c):
                base = r * _L
                ct = base // 128
                cc = base % 128
                cols = idx_v[pl.ds(base, _L)]
                for dd in range(_DCH):
                    d = c * _DCH + dd
                    off = (dd // 8) * 8192 + (dd % 8) * 128 + cc
                    q_v[pl.ds(ct * 1024 + off, _L)] = plsc.load_gather(
                        et_v, [cols + d * _NE])

            pltpu.sync_copy(q_v, out_hbm.at[b, pl.ds(c * _DCH * _BR,
                                                     _DCH * _BR)])

    return _sc_gather


def kernel(inputs, embeddings):
    orig_shape = inputs.shape
    xt3 = jnp.swapaxes(inputs, 1, 2)
    idx3, loss11 = _tc_call(xt3, embeddings)
    idx2d = idx3.reshape(_B, _BR)
    etf = embeddings.T.reshape(-1)
    qt = _sc_gather_call()(etf, idx2d)
    quantized = jnp.transpose(qt.reshape(_B, 8, 8, 8, 128),
                              (0, 2, 4, 1, 3)).reshape(orig_shape)
    loss = loss11[0, 0]
    encoding_indices = idx2d.reshape(orig_shape[:-1])
    return (quantized, loss, encoding_indices)


# speedup vs baseline: 1.1620x; 1.0280x over previous
import functools

import jax
import jax.numpy as jnp
from jax import lax
from jax.experimental import pallas as pl
from jax.experimental.pallas import tpu as pltpu
from jax.experimental.pallas import tpu_sc as plsc

_NE = 1024
_D = 64
_B = 32
_BR = 1024
_NROWS = _B * _BR
_NC, _NS = 2, 16
_L = 16
_DCH = 16


_BB = 4


def _tc_body(xt_ref, e_ref, idx_ref, loss_ref):
    i = pl.program_id(0)
    part = 0.0
    for j in range(_BB):
        xt = xt_ref[j]
        ssq = jnp.sum(xt * xt, axis=0, keepdims=True)
        norm = jnp.sqrt(ssq)
        inv = 1.0 / jnp.maximum(norm, 1e-12)
        xnt = xt * inv

        s = lax.dot_general(e_ref[...], xnt, (((1,), (0,)), ((), ())),
                            preferred_element_type=jnp.float32)
        m = jnp.max(s, axis=0, keepdims=True)
        idx = jnp.argmax(s, axis=0)[None].astype(jnp.int32)
        idx_ref[j] = idx

        xnsq = ssq * (inv * inv)
        part += jnp.sum(xnsq - 2.0 * m + 1.0) * (1.0 / (_NROWS * _D))

    @pl.when(i == 0)
    def _():
        loss_ref[0, 0] = 0.0

    loss_ref[0, 0] += part


_tc_call = pl.pallas_call(
    _tc_body,
    grid=(_B // _BB,),
    in_specs=[
        pl.BlockSpec((_BB, _D, _BR), lambda i: (i, 0, 0)),
        pl.BlockSpec((_NE, _D), lambda i: (0, 0)),
    ],
    out_specs=[
        pl.BlockSpec((_BB, 1, _BR), lambda i: (i, 0, 0)),
        pl.BlockSpec(memory_space=pltpu.SMEM, block_shape=(1, 1),
                     index_map=lambda i: (0, 0)),
    ],
    out_shape=[
        jax.ShapeDtypeStruct((_B, 1, _BR), jnp.int32),
        jax.ShapeDtypeStruct((1, 1), jnp.float32),
    ],
    compiler_params=pltpu.CompilerParams(dimension_semantics=("arbitrary",)),
)


@functools.cache
def _sc_gather_call():
    @functools.partial(
        pl.kernel,
        mesh=plsc.VectorSubcoreMesh(core_axis_name="c", subcore_axis_name="s"),
        out_type=jax.ShapeDtypeStruct((_B, _D * _BR), jnp.float32),
        scratch_types=[
            pltpu.VMEM((_D * _NE,), jnp.float32),
            pltpu.VMEM((_BR,), jnp.int32),
            pltpu.VMEM((_DCH * _BR,), jnp.float32),
        ],
        compiler_params=pltpu.CompilerParams(use_tc_tiling_on_sc=False,
                                             needs_layout_passes=False),
    )
    def _sc_gather(etf_hbm, idx_hbm, out_hbm, et_v, idx_v, q_v):
        b = lax.axis_index("s") * _NC + lax.axis_index("c")
        pltpu.sync_copy(etf_hbm, et_v)
        pltpu.sync_copy(idx_hbm.at[b], idx_v)

        for c in range(_D // _DCH):
            @plsc.parallel_loop(0, _BR // _L, 1, unroll=8)
            def _(r, c=c):
                base = r * _L
                ct = base // 128
                cc = base % 128
                cols = idx_v[pl.ds(base, _L)]
                for dd in range(_DCH):
                    d = c * _DCH + dd
                    off = (dd // 8) * 8192 + (dd % 8) * 128 + cc
                    q_v[pl.ds(ct * 1024 + off, _L)] = plsc.load_gather(
                        et_v, [cols + d * _NE])

            pltpu.sync_copy(q_v, out_hbm.at[b, pl.ds(c * _DCH * _BR,
                                                     _DCH * _BR)])

    return _sc_gather


def kernel(inputs, embeddings):
    orig_shape = inputs.shape
    xt3 = jnp.swapaxes(inputs, 1, 2)
    idx3, loss11 = _tc_call(xt3, embeddings)
    idx2d = idx3.reshape(_B, _BR)
    etf = embeddings.T.reshape(-1)
    qt = _sc_gather_call()(etf, idx2d)
    quantized = jnp.transpose(qt.reshape(_B, 8, 8, 8, 128),
                              (0, 2, 4, 1, 3)).reshape(orig_shape)
    loss = loss11[0, 0]
    encoding_indices = idx2d.reshape(orig_shape[:-1])
    return (quantized, loss, encoding_indices)


# speedup vs baseline: 1.1786x; 1.0143x over previous
import functools

import jax
import jax.numpy as jnp
from jax import lax
from jax.experimental import pallas as pl
from jax.experimental.pallas import tpu as pltpu
from jax.experimental.pallas import tpu_sc as plsc

_NE = 1024
_D = 64
_B = 32
_BR = 1024
_NROWS = _B * _BR
_NC, _NS = 2, 16
_L = 16
_DCH = 16


_BB = 8


def _tc_body(xt_ref, e_ref, idx_ref, loss_ref):
    i = pl.program_id(0)
    part = 0.0
    for j in range(_BB):
        xt = xt_ref[j]
        ssq = jnp.sum(xt * xt, axis=0, keepdims=True)
        norm = jnp.sqrt(ssq)
        inv = 1.0 / jnp.maximum(norm, 1e-12)
        xnt = xt * inv

        s = lax.dot_general(e_ref[...], xnt, (((1,), (0,)), ((), ())),
                            preferred_element_type=jnp.float32)
        m = jnp.max(s, axis=0, keepdims=True)
        idx = jnp.argmax(s, axis=0)[None].astype(jnp.int32)
        idx_ref[j] = idx

        xnsq = ssq * (inv * inv)
        part += jnp.sum(xnsq - 2.0 * m + 1.0) * (1.0 / (_NROWS * _D))

    @pl.when(i == 0)
    def _():
        loss_ref[0, 0] = 0.0

    loss_ref[0, 0] += part


_tc_call = pl.pallas_call(
    _tc_body,
    grid=(_B // _BB,),
    in_specs=[
        pl.BlockSpec((_BB, _D, _BR), lambda i: (i, 0, 0)),
        pl.BlockSpec((_NE, _D), lambda i: (0, 0)),
    ],
    out_specs=[
        pl.BlockSpec((_BB, 1, _BR), lambda i: (i, 0, 0)),
        pl.BlockSpec(memory_space=pltpu.SMEM, block_shape=(1, 1),
                     index_map=lambda i: (0, 0)),
    ],
    out_shape=[
        jax.ShapeDtypeStruct((_B, 1, _BR), jnp.int32),
        jax.ShapeDtypeStruct((1, 1), jnp.float32),
    ],
    compiler_params=pltpu.CompilerParams(dimension_semantics=("arbitrary",)),
)


@functools.cache
def _sc_gather_call():
    @functools.partial(
        pl.kernel,
        mesh=plsc.VectorSubcoreMesh(core_axis_name="c", subcore_axis_name="s"),
        out_type=jax.ShapeDtypeStruct((_B, _D * _BR), jnp.float32),
        scratch_types=[
            pltpu.VMEM((_D * _NE,), jnp.float32),
            pltpu.VMEM((_BR,), jnp.int32),
            pltpu.VMEM((_DCH * _BR,), jnp.float32),
        ],
        compiler_params=pltpu.CompilerParams(use_tc_tiling_on_sc=False,
                                             needs_layout_passes=False),
    )
    def _sc_gather(etf_hbm, idx_hbm, out_hbm, et_v, idx_v, q_v):
        b = lax.axis_index("s") * _NC + lax.axis_index("c")
        pltpu.sync_copy(etf_hbm, et_v)
        pltpu.sync_copy(idx_hbm.at[b], idx_v)

        for c in range(_D // _DCH):
            @plsc.parallel_loop(0, _BR // _L, 1, unroll=8)
            def _(r, c=c):
                base = r * _L
                ct = base // 128
                cc = base % 128
                cols = idx_v[pl.ds(base, _L)]
                for dd in range(_DCH):
                    d = c * _DCH + dd
                    off = (dd // 8) * 8192 + (dd % 8) * 128 + cc
                    q_v[pl.ds(ct * 1024 + off, _L)] = plsc.load_gather(
                        et_v, [cols + d * _NE])

            pltpu.sync_copy(q_v, out_hbm.at[b, pl.ds(c * _DCH * _BR,
                                                     _DCH * _BR)])

    return _sc_gather


def kernel(inputs, embeddings):
    orig_shape = inputs.shape
    xt3 = jnp.swapaxes(inputs, 1, 2)
    idx3, loss11 = _tc_call(xt3, embeddings)
    idx2d = idx3.reshape(_B, _BR)
    etf = embeddings.T.reshape(-1)
    qt = _sc_gather_call()(etf, idx2d)
    quantized = jnp.transpose(qt.reshape(_B, 8, 8, 8, 128),
                              (0, 2, 4, 1, 3)).reshape(orig_shape)
    loss = loss11[0, 0]
    encoding_indices = idx2d.reshape(orig_shape[:-1])
    return (quantized, loss, encoding_indices)


# speedup vs baseline: 1.2128x; 1.0291x over previous
import functools

import jax
import jax.numpy as jnp
from jax import lax
from jax.experimental import pallas as pl
from jax.experimental.pallas import tpu as pltpu
from jax.experimental.pallas import tpu_sc as plsc

_NE = 1024
_D = 64
_B = 32
_BR = 1024
_NROWS = _B * _BR
_NC, _NS = 2, 16
_L = 16
_DCH = 16


_BB = 8


def _tc_body(xt_ref, e_ref, idx_ref, loss_ref):
    i = pl.program_id(0)
    part = 0.0
    for j in range(_BB):
        xt = xt_ref[j]
        ssq = jnp.sum(xt * xt, axis=0, keepdims=True)
        norm = jnp.sqrt(ssq)
        inv = 1.0 / jnp.maximum(norm, 1e-12)
        xnt = xt * inv

        s = lax.dot_general(e_ref[...], xnt, (((0,), (0,)), ((), ())),
                            preferred_element_type=jnp.float32)
        m = jnp.max(s, axis=0, keepdims=True)
        idx = jnp.argmax(s, axis=0)[None].astype(jnp.int32)
        idx_ref[j] = idx

        xnsq = ssq * (inv * inv)
        part += jnp.sum(xnsq - 2.0 * m + 1.0) * (1.0 / (_NROWS * _D))

    @pl.when(i == 0)
    def _():
        loss_ref[0, 0] = 0.0

    loss_ref[0, 0] += part


_tc_call = pl.pallas_call(
    _tc_body,
    grid=(_B // _BB,),
    in_specs=[
        pl.BlockSpec((_BB, _D, _BR), lambda i: (i, 0, 0)),
        pl.BlockSpec((_D, _NE), lambda i: (0, 0)),
    ],
    out_specs=[
        pl.BlockSpec((_BB, 1, _BR), lambda i: (i, 0, 0)),
        pl.BlockSpec(memory_space=pltpu.SMEM, block_shape=(1, 1),
                     index_map=lambda i: (0, 0)),
    ],
    out_shape=[
        jax.ShapeDtypeStruct((_B, 1, _BR), jnp.int32),
        jax.ShapeDtypeStruct((1, 1), jnp.float32),
    ],
    compiler_params=pltpu.CompilerParams(dimension_semantics=("arbitrary",)),
)


@functools.cache
def _sc_gather_call():
    @functools.partial(
        pl.kernel,
        mesh=plsc.VectorSubcoreMesh(core_axis_name="c", subcore_axis_name="s"),
        out_type=jax.ShapeDtypeStruct((_B, _D * _BR), jnp.float32),
        scratch_types=[
            pltpu.VMEM((_D * _NE,), jnp.float32),
            pltpu.VMEM((_BR,), jnp.int32),
            pltpu.VMEM((_DCH * _BR,), jnp.float32),
        ],
        compiler_params=pltpu.CompilerParams(use_tc_tiling_on_sc=False,
                                             needs_layout_passes=False),
    )
    def _sc_gather(etf_hbm, idx_hbm, out_hbm, et_v, idx_v, q_v):
        b = lax.axis_index("s") * _NC + lax.axis_index("c")
        pltpu.sync_copy(etf_hbm, et_v)
        pltpu.sync_copy(idx_hbm.at[b], idx_v)

        for c in range(_D // _DCH):
            @plsc.parallel_loop(0, _BR // _L, 1, unroll=8)
            def _(r, c=c):
                base = r * _L
                ct = base // 128
                cc = base % 128
                cols = idx_v[pl.ds(base, _L)]
                for dd in range(_DCH):
                    d = c * _DCH + dd
                    off = (dd // 8) * 8192 + (dd % 8) * 128 + cc
                    q_v[pl.ds(ct * 1024 + off, _L)] = plsc.load_gather(
                        et_v, [cols + d * _NE])

            pltpu.sync_copy(q_v, out_hbm.at[b, pl.ds(c * _DCH * _BR,
                                                     _DCH * _BR)])

    return _sc_gather


def kernel(inputs, embeddings):
    orig_shape = inputs.shape
    xt3 = jnp.swapaxes(inputs, 1, 2)
    idx3, loss11 = _tc_call(xt3, embeddings.T)
    idx2d = idx3.reshape(_B, _BR)
    etf = embeddings.T.reshape(-1)
    qt = _sc_gather_call()(etf, idx2d)
    quantized = jnp.transpose(qt.reshape(_B, 8, 8, 8, 128),
                              (0, 2, 4, 1, 3)).reshape(orig_shape)
    loss = loss11[0, 0]
    encoding_indices = idx2d.reshape(orig_shape[:-1])
    return (quantized, loss, encoding_indices)
